# Initial kernel scaffold; baseline (speedup 1.0000x reference)
import jax
import jax.numpy as jnp
from jax.experimental import pallas as pl


def kernel(x, edge_index, batch, W1, b1, W2, b2, FW1, Fb1, FW2, Fb2):
    raise NotImplementedError("write your pallas kernel here")


# speedup vs baseline: 17.2288x; 17.2288x over previous
import functools

import jax
import jax.numpy as jnp
from jax import lax
from jax.experimental import pallas as pl
from jax.experimental.pallas import tpu as pltpu
from jax.experimental.pallas import tpu_sc as plsc

_N = 10000
_E = 320000
_D = 128
_G = 64
_NC = 2
_NS = 16
_NW = _NC * _NS
_EW = _E // _NW
_K = 100
_NCH = _EW // _K
_RT = _N // _NS
_DW = 16

_BR = 400
_GR = _N // _BR


def _sc_mesh():
    return plsc.VectorSubcoreMesh(core_axis_name="c", subcore_axis_name="s")


def _sc_degree(dst_r, ones_hbm, zrows):
    @functools.partial(
        pl.kernel,
        out_type=jax.ShapeDtypeStruct((_NC, _NS, _RT, _D), jnp.float32),
        mesh=_sc_mesh(),
        scratch_types=[
            pltpu.VMEM((_NCH, _K), jnp.int32),
            pltpu.VMEM((_K, _D), jnp.float32),
            pltpu.VMEM_SHARED((_N, _D), jnp.float32),
        ],
    )
    def k(dst_hbm, ones_h, z_hbm, out_hbm, dst_v, ones_v, acc):
        c = lax.axis_index("c")
        s = lax.axis_index("s")
        w = c * _NS + s
        pltpu.sync_copy(z_hbm, acc.at[pl.ds(s * _RT, _RT)])
        pltpu.sync_copy(ones_h, ones_v)
        pltpu.sync_copy(dst_hbm.at[w], dst_v)
        plsc.subcore_barrier()

        def body(j, carry):
            pltpu.sync_copy(ones_v, acc.at[dst_v.at[j]], add=True)
            return carry

        lax.fori_loop(0, _NCH, body, 0)
        plsc.subcore_barrier()
        pltpu.sync_copy(acc.at[pl.ds(s * _RT, _RT)], out_hbm.at[c].at[s])

    return k(dst_r, ones_hbm, zrows).reshape(_NC, _N, _D)


def _sc_msgpass(y, src_r, dst_r, zrows):
    @functools.partial(
        pl.kernel,
        out_type=jax.ShapeDtypeStruct((_NC, _NS, _RT, _D), jnp.float32),
        mesh=_sc_mesh(),
        scratch_types=[
            pltpu.VMEM((_NCH, _K), jnp.int32),
            pltpu.VMEM((_NCH, _K), jnp.int32),
            pltpu.VMEM((_K, _D), jnp.float32),
            pltpu.VMEM_SHARED((_N, _D), jnp.float32),
            pltpu.SemaphoreType.DMA,
        ],
    )
    def k(y_hbm, src_hbm, dst_hbm, z_hbm, out_hbm, src_v, dst_v, rows, acc, sem):
        c = lax.axis_index("c")
        s = lax.axis_index("s")
        w = c * _NS + s
        pltpu.sync_copy(z_hbm, acc.at[pl.ds(s * _RT, _RT)])
        pltpu.sync_copy(src_hbm.at[w], src_v)
        pltpu.sync_copy(dst_hbm.at[w], dst_v)
        plsc.subcore_barrier()

        def body(j, carry):
            pltpu.async_copy(y_hbm.at[src_v.at[j]], rows, sem).wait()
            pltpu.sync_copy(rows, acc.at[dst_v.at[j]], add=True)
            return carry

        lax.fori_loop(0, _NCH, body, 0)
        plsc.subcore_barrier()
        pltpu.sync_copy(acc.at[pl.ds(s * _RT, _RT)], out_hbm.at[c].at[s])

    return k(y, src_r, dst_r, zrows).reshape(_NC, _N, _D)


def _dis_block(degp_blk):
    deg = degp_blk[0][:, 0:1] + degp_blk[1][:, 0:1] + 1.0
    return 1.0 / jnp.sqrt(deg)


def _tc_y1(x, W1, degp):
    def body(x_ref, w_ref, degp_ref, y_ref):
        dis = _dis_block(degp_ref)
        xw = jnp.dot(x_ref[...], w_ref[...], preferred_element_type=jnp.float32)
        y_ref[...] = xw * dis

    return pl.pallas_call(
        body,
        grid=(_GR,),
        in_specs=[
            pl.BlockSpec((_BR, _D), lambda i: (i, 0)),
            pl.BlockSpec((_D, _D), lambda i: (0, 0)),
            pl.BlockSpec((_NC, _BR, _DW), lambda i: (0, i, 0)),
        ],
        out_specs=pl.BlockSpec((_BR, _D), lambda i: (i, 0)),
        out_shape=jax.ShapeDtypeStruct((_N, _D), jnp.float32),
    )(x, W1, degp)


def _tc_mid(S1, y1, degp, W2, b1r):
    def body(s_ref, y1_ref, degp_ref, w2_ref, b1_ref, y2_ref):
        dis = _dis_block(degp_ref)
        t = s_ref[0] + s_ref[1] + y1_ref[...]
        h1 = jnp.maximum(dis * t + b1_ref[...], 0.0)
        y2_ref[...] = jnp.dot(h1, w2_ref[...],
                              preferred_element_type=jnp.float32) * dis

    return pl.pallas_call(
        body,
        grid=(_GR,),
        in_specs=[
            pl.BlockSpec((_NC, _BR, _D), lambda i: (0, i, 0)),
            pl.BlockSpec((_BR, _D), lambda i: (i, 0)),
            pl.BlockSpec((_NC, _BR, _DW), lambda i: (0, i, 0)),
            pl.BlockSpec((_D, _D), lambda i: (0, 0)),
            pl.BlockSpec((1, _D), lambda i: (0, 0)),
        ],
        out_specs=pl.BlockSpec((_BR, _D), lambda i: (i, 0)),
        out_shape=jax.ShapeDtypeStruct((_N, _D), jnp.float32),
    )(S1, y1, degp, W2, b1r)


def _tc_final(S2, y2, degp, batch_r, b2r, FW1, Fb1r, FW2r, Fb2r):
    def body(s_ref, y2_ref, degp_ref, batch_ref, b2_ref, fw1_ref, fb1_ref,
             fw2_ref, fb2_ref, out_ref, sums, cnt):
        i = pl.program_id(0)

        @pl.when(i == 0)
        def _():
            sums[...] = jnp.zeros_like(sums)
            cnt[...] = jnp.zeros_like(cnt)

        dis = _dis_block(degp_ref)
        t = s_ref[0] + s_ref[1] + y2_ref[...]
        h2 = jnp.maximum(dis * t + b2_ref[...], 0.0)
        b = batch_ref[0, 0, :]
        gid = lax.broadcasted_iota(jnp.int32, (_G, _BR), 0)
        onehot = (gid == b[None, :]).astype(jnp.float32)
        sums[...] += jnp.dot(onehot, h2, preferred_element_type=jnp.float32, precision=lax.Precision.HIGHEST)
        cnt[...] += jnp.sum(onehot, axis=1, keepdims=True)

        @pl.when(i == _GR - 1)
        def _():
            pooled = sums[...] / jnp.maximum(cnt[...], 1.0)
            hid = jnp.maximum(
                jnp.dot(pooled, fw1_ref[...],
                        preferred_element_type=jnp.float32) + fb1_ref[...],
                0.0)
            hidb = hid.astype(jnp.bfloat16).astype(jnp.float32)
            fw2b = fw2_ref[...].astype(jnp.bfloat16).astype(jnp.float32)
            out_ref[...] = (jnp.sum(hidb * fw2b, axis=1, keepdims=True)
                            + fb2_ref[...])

    return pl.pallas_call(
        body,
        grid=(_GR,),
        in_specs=[
            pl.BlockSpec((_NC, _BR, _D), lambda i: (0, i, 0)),
            pl.BlockSpec((_BR, _D), lambda i: (i, 0)),
            pl.BlockSpec((_NC, _BR, _DW), lambda i: (0, i, 0)),
            pl.BlockSpec((1, 1, _BR), lambda i: (i, 0, 0)),
            pl.BlockSpec((1, _D), lambda i: (0, 0)),
            pl.BlockSpec((_D, _G), lambda i: (0, 0)),
            pl.BlockSpec((1, _G), lambda i: (0, 0)),
            pl.BlockSpec((1, _G), lambda i: (0, 0)),
            pl.BlockSpec((1, 1), lambda i: (0, 0)),
        ],
        out_specs=pl.BlockSpec((_G, 1), lambda i: (0, 0)),
        out_shape=jax.ShapeDtypeStruct((_G, 1), jnp.float32),
        scratch_shapes=[
            pltpu.VMEM((_G, _D), jnp.float32),
            pltpu.VMEM((_G, _D), jnp.float32),
        ],
    )(S2, y2, degp, batch_r, b2r, FW1, Fb1r, FW2r, Fb2r)


def kernel(x, edge_index, batch, W1, b1, W2, b2, FW1, Fb1, FW2, Fb2):
    src_r = edge_index[0].reshape(_NW, _NCH, _K)
    dst_r = edge_index[1].reshape(_NW, _NCH, _K)
    batch_r = batch.reshape(_GR, 1, _BR)
    zrows = jnp.zeros((_RT, _D), jnp.float32)
    ones_rows = jnp.ones((_K, _D), jnp.float32)
    b1r = b1.reshape(1, _D)
    b2r = b2.reshape(1, _D)
    Fb1r = Fb1.reshape(1, _G)
    FW2r = FW2.reshape(1, _G)
    Fb2r = Fb2.reshape(1, 1)

    degp = _sc_degree(dst_r, ones_rows, zrows)[:, :, :_DW]
    y1 = _tc_y1(x, W1, degp)
    S1 = _sc_msgpass(y1, src_r, dst_r, zrows)
    y2 = _tc_mid(S1, y1, degp, W2, b1r)
    S2 = _sc_msgpass(y2, src_r, dst_r, zrows)
    out = _tc_final(S2, y2, degp, batch_r, b2r, FW1, Fb1r, FW2r, Fb2r)
    return jnp.squeeze(out, axis=-1)


# speedup vs baseline: 20.3990x; 1.1840x over previous
import functools

import jax
import jax.numpy as jnp
from jax import lax
from jax.experimental import pallas as pl
from jax.experimental.pallas import tpu as pltpu
from jax.experimental.pallas import tpu_sc as plsc

_N = 10000
_E = 320000
_D = 128
_G = 64
_NC = 2
_NS = 16
_NW = _NC * _NS
_EW = _E // _NW
_K = 100
_NCH = _EW // _K
_NB = 2
_NCHB = _NCH // _NB
_RT = _N // _NS
_DW = 16

_BR = 400
_GR = _N // _BR


def _sc_mesh():
    return plsc.VectorSubcoreMesh(core_axis_name="c", subcore_axis_name="s")


def _sc_degree(dst_r, ones_hbm, zrows):
    @functools.partial(
        pl.kernel,
        out_type=jax.ShapeDtypeStruct((_NC, _NS, _RT, _D), jnp.float32),
        mesh=_sc_mesh(),
        scratch_types=[
            pltpu.VMEM((_NCH, _K), jnp.int32),
            pltpu.VMEM((_K, _D), jnp.float32),
            pltpu.VMEM_SHARED((_N, _D), jnp.float32),
        ],
    )
    def k(dst_hbm, ones_h, z_hbm, out_hbm, dst_v, ones_v, acc):
        c = lax.axis_index("c")
        s = lax.axis_index("s")
        w = c * _NS + s
        pltpu.sync_copy(z_hbm, acc.at[pl.ds(s * _RT, _RT)])
        pltpu.sync_copy(ones_h, ones_v)
        pltpu.sync_copy(dst_hbm.at[w], dst_v)
        plsc.subcore_barrier()

        def body(j, carry):
            pltpu.sync_copy(ones_v, acc.at[dst_v.at[j]], add=True)
            return carry

        lax.fori_loop(0, _NCH, body, 0)
        plsc.subcore_barrier()
        pltpu.sync_copy(acc.at[pl.ds(s * _RT, _RT)], out_hbm.at[c].at[s])

    return k(dst_r, ones_hbm, zrows).reshape(_NC, _N, _D)


def _sc_msgpass(y, src_r, dst_r, zrows):
    @functools.partial(
        pl.kernel,
        out_type=jax.ShapeDtypeStruct((_NC, _NS, _RT, _D), jnp.float32),
        mesh=_sc_mesh(),
        scratch_types=[
            pltpu.VMEM((_NCHB, _K), jnp.int32),
            pltpu.VMEM((_NCHB, _K), jnp.int32),
            pltpu.VMEM((_K, _D), jnp.float32),
            pltpu.VMEM((_K, _D), jnp.float32),
            pltpu.VMEM_SHARED((_N, _D), jnp.float32),
            pltpu.SemaphoreType.DMA,
            pltpu.SemaphoreType.DMA,
        ],
    )
    def k(y_hbm, src_hbm, dst_hbm, z_hbm, out_hbm, src_v, dst_v, rows0, rows1,
          acc, sem0, sem1):
        c = lax.axis_index("c")
        s = lax.axis_index("s")
        w = c * _NS + s
        pltpu.sync_copy(z_hbm, acc.at[pl.ds(s * _RT, _RT)])
        plsc.subcore_barrier()

        for t in range(_NB):
            pltpu.sync_copy(src_hbm.at[w].at[t], src_v)
            pltpu.sync_copy(dst_hbm.at[w].at[t], dst_v)
            pltpu.async_copy(y_hbm.at[src_v.at[0]], rows0, sem0).wait()

            def body(jj, carry):
                j0 = 2 * jj
                j1 = j0 + 1
                jn = jnp.minimum(j0 + 2, _NCHB - 1)
                cp1 = pltpu.async_copy(y_hbm.at[src_v.at[j1]], rows1, sem1)
                pltpu.sync_copy(rows0, acc.at[dst_v.at[j0]], add=True)
                cp1.wait()
                cp0 = pltpu.async_copy(y_hbm.at[src_v.at[jn]], rows0, sem0)
                pltpu.sync_copy(rows1, acc.at[dst_v.at[j1]], add=True)
                cp0.wait()
                return carry

            lax.fori_loop(0, _NCHB // 2, body, 0)
        plsc.subcore_barrier()
        pltpu.sync_copy(acc.at[pl.ds(s * _RT, _RT)], out_hbm.at[c].at[s])

    return k(y, src_r, dst_r, zrows).reshape(_NC, _N, _D)


def _dis_block(degp_blk):
    deg = degp_blk[0][:, 0:1] + degp_blk[1][:, 0:1] + 1.0
    return 1.0 / jnp.sqrt(deg)


def _tc_y1(x, W1, degp):
    def body(x_ref, w_ref, degp_ref, y_ref):
        dis = _dis_block(degp_ref)
        xw = jnp.dot(x_ref[...], w_ref[...], preferred_element_type=jnp.float32)
        y_ref[...] = xw * dis

    return pl.pallas_call(
        body,
        grid=(_GR,),
        in_specs=[
            pl.BlockSpec((_BR, _D), lambda i: (i, 0)),
            pl.BlockSpec((_D, _D), lambda i: (0, 0)),
            pl.BlockSpec((_NC, _BR, _DW), lambda i: (0, i, 0)),
        ],
        out_specs=pl.BlockSpec((_BR, _D), lambda i: (i, 0)),
        out_shape=jax.ShapeDtypeStruct((_N, _D), jnp.float32),
    )(x, W1, degp)


def _tc_mid(S1, y1, degp, W2, b1r):
    def body(s_ref, y1_ref, degp_ref, w2_ref, b1_ref, y2_ref):
        dis = _dis_block(degp_ref)
        t = s_ref[0] + s_ref[1] + y1_ref[...]
        h1 = jnp.maximum(dis * t + b1_ref[...], 0.0)
        y2_ref[...] = jnp.dot(h1, w2_ref[...],
                              preferred_element_type=jnp.float32) * dis

    return pl.pallas_call(
        body,
        grid=(_GR,),
        in_specs=[
            pl.BlockSpec((_NC, _BR, _D), lambda i: (0, i, 0)),
            pl.BlockSpec((_BR, _D), lambda i: (i, 0)),
            pl.BlockSpec((_NC, _BR, _DW), lambda i: (0, i, 0)),
            pl.BlockSpec((_D, _D), lambda i: (0, 0)),
            pl.BlockSpec((1, _D), lambda i: (0, 0)),
        ],
        out_specs=pl.BlockSpec((_BR, _D), lambda i: (i, 0)),
        out_shape=jax.ShapeDtypeStruct((_N, _D), jnp.float32),
    )(S1, y1, degp, W2, b1r)


def _tc_final(S2, y2, degp, batch_r, b2r, FW1, Fb1r, FW2r, Fb2r):
    def body(s_ref, y2_ref, degp_ref, batch_ref, b2_ref, fw1_ref, fb1_ref,
             fw2_ref, fb2_ref, out_ref, sums, cnt):
        i = pl.program_id(0)

        @pl.when(i == 0)
        def _():
            sums[...] = jnp.zeros_like(sums)
            cnt[...] = jnp.zeros_like(cnt)

        dis = _dis_block(degp_ref)
        t = s_ref[0] + s_ref[1] + y2_ref[...]
        h2 = jnp.maximum(dis * t + b2_ref[...], 0.0)
        b = batch_ref[0, 0, :]
        gid = lax.broadcasted_iota(jnp.int32, (_G, _BR), 0)
        onehot = (gid == b[None, :]).astype(jnp.float32)
        sums[...] += jnp.dot(onehot, h2, preferred_element_type=jnp.float32, precision=lax.Precision.HIGHEST)
        cnt[...] += jnp.sum(onehot, axis=1, keepdims=True)

        @pl.when(i == _GR - 1)
        def _():
            pooled = sums[...] / jnp.maximum(cnt[...], 1.0)
            hid = jnp.maximum(
                jnp.dot(pooled, fw1_ref[...],
                        preferred_element_type=jnp.float32) + fb1_ref[...],
                0.0)
            hidb = hid.astype(jnp.bfloat16).astype(jnp.float32)
            fw2b = fw2_ref[...].astype(jnp.bfloat16).astype(jnp.float32)
            out_ref[...] = (jnp.sum(hidb * fw2b, axis=1, keepdims=True)
                            + fb2_ref[...])

    return pl.pallas_call(
        body,
        grid=(_GR,),
        in_specs=[
            pl.BlockSpec((_NC, _BR, _D), lambda i: (0, i, 0)),
            pl.BlockSpec((_BR, _D), lambda i: (i, 0)),
            pl.BlockSpec((_NC, _BR, _DW), lambda i: (0, i, 0)),
            pl.BlockSpec((1, 1, _BR), lambda i: (i, 0, 0)),
            pl.BlockSpec((1, _D), lambda i: (0, 0)),
            pl.BlockSpec((_D, _G), lambda i: (0, 0)),
            pl.BlockSpec((1, _G), lambda i: (0, 0)),
            pl.BlockSpec((1, _G), lambda i: (0, 0)),
            pl.BlockSpec((1, 1), lambda i: (0, 0)),
        ],
        out_specs=pl.BlockSpec((_G, 1), lambda i: (0, 0)),
        out_shape=jax.ShapeDtypeStruct((_G, 1), jnp.float32),
        scratch_shapes=[
            pltpu.VMEM((_G, _D), jnp.float32),
            pltpu.VMEM((_G, _D), jnp.float32),
        ],
    )(S2, y2, degp, batch_r, b2r, FW1, Fb1r, FW2r, Fb2r)


def kernel(x, edge_index, batch, W1, b1, W2, b2, FW1, Fb1, FW2, Fb2):
    src_r = edge_index[0].reshape(_NW, _NB, _NCHB, _K)
    dst_r = edge_index[1].reshape(_NW, _NB, _NCHB, _K)
    dst_deg = edge_index[1].reshape(_NW, _NCH, _K)
    batch_r = batch.reshape(_GR, 1, _BR)
    zrows = jnp.zeros((_RT, _D), jnp.float32)
    ones_rows = jnp.ones((_K, _D), jnp.float32)
    b1r = b1.reshape(1, _D)
    b2r = b2.reshape(1, _D)
    Fb1r = Fb1.reshape(1, _G)
    FW2r = FW2.reshape(1, _G)
    Fb2r = Fb2.reshape(1, 1)

    degp = _sc_degree(dst_deg, ones_rows, zrows)[:, :, :_DW]
    y1 = _tc_y1(x, W1, degp)
    S1 = _sc_msgpass(y1, src_r, dst_r, zrows)
    y2 = _tc_mid(S1, y1, degp, W2, b1r)
    S2 = _sc_msgpass(y2, src_r, dst_r, zrows)
    out = _tc_final(S2, y2, degp, batch_r, b2r, FW1, Fb1r, FW2r, Fb2r)
    return jnp.squeeze(out, axis=-1)


# speedup vs baseline: 20.9496x; 1.0270x over previous
import functools

import jax
import jax.numpy as jnp
from jax import lax
from jax.experimental import pallas as pl
from jax.experimental.pallas import tpu as pltpu
from jax.experimental.pallas import tpu_sc as plsc

_N = 10000
_E = 320000
_D = 128
_G = 64
_NC = 2
_NS = 16
_NW = _NC * _NS
_EW = _E // _NW
_K = 50
_NCH = _EW // _K
_NB = 5
_NCHB = _NCH // _NB
_NG = _NCHB // 4
_RT = _N // _NS
_DW = 16

_BR = 400
_GR = _N // _BR


def _sc_mesh():
    return plsc.VectorSubcoreMesh(core_axis_name="c", subcore_axis_name="s")


def _sc_degree(dst_r, ones_hbm, zrows):
    @functools.partial(
        pl.kernel,
        out_type=jax.ShapeDtypeStruct((_NC, _NS, _RT, _D), jnp.float32),
        mesh=_sc_mesh(),
        scratch_types=[
            pltpu.VMEM((_NCH, _K), jnp.int32),
            pltpu.VMEM((_K, _D), jnp.float32),
            pltpu.VMEM_SHARED((_N, _D), jnp.float32),
        ],
    )
    def k(dst_hbm, ones_h, z_hbm, out_hbm, dst_v, ones_v, acc):
        c = lax.axis_index("c")
        s = lax.axis_index("s")
        w = c * _NS + s
        pltpu.sync_copy(z_hbm, acc.at[pl.ds(s * _RT, _RT)])
        pltpu.sync_copy(ones_h, ones_v)
        pltpu.sync_copy(dst_hbm.at[w], dst_v)
        plsc.subcore_barrier()

        def body(j, carry):
            pltpu.sync_copy(ones_v, acc.at[dst_v.at[j]], add=True)
            return carry

        lax.fori_loop(0, _NCH, body, 0)
        plsc.subcore_barrier()
        pltpu.sync_copy(acc.at[pl.ds(s * _RT, _RT)], out_hbm.at[c].at[s])

    return k(dst_r, ones_hbm, zrows).reshape(_NC, _N, _D)


def _sc_msgpass(y, src_r, dst_r, zrows):
    @functools.partial(
        pl.kernel,
        out_type=jax.ShapeDtypeStruct((_NC, _NS, _RT, _D), jnp.float32),
        mesh=_sc_mesh(),
        scratch_types=[
            pltpu.VMEM((_NCHB, _K), jnp.int32),
            pltpu.VMEM((_NCHB, _K), jnp.int32),
            pltpu.VMEM((_K, _D), jnp.float32),
            pltpu.VMEM((_K, _D), jnp.float32),
            pltpu.VMEM((_K, _D), jnp.float32),
            pltpu.VMEM((_K, _D), jnp.float32),
            pltpu.VMEM_SHARED((_N, _D), jnp.float32),
            [pltpu.SemaphoreType.DMA] * 4,
            [pltpu.SemaphoreType.DMA] * 4,
        ],
    )
    def k(y_hbm, src_hbm, dst_hbm, z_hbm, out_hbm, src_v, dst_v, r0, r1, r2,
          r3, acc, gsem, ssem):
        c = lax.axis_index("c")
        s = lax.axis_index("s")
        w = c * _NS + s
        rowsl = (r0, r1, r2, r3)
        pltpu.sync_copy(z_hbm, acc.at[pl.ds(s * _RT, _RT)])
        plsc.subcore_barrier()

        def g_issue(j, b):
            pltpu.async_copy(y_hbm.at[src_v.at[j]], rowsl[b], gsem[b])

        def g_wait(b):
            pltpu.make_async_copy(y_hbm.at[src_v.at[0]], rowsl[b],
                                  gsem[b]).wait()

        def s_issue(j, b):
            pltpu.async_copy(rowsl[b], acc.at[dst_v.at[j]], ssem[b], add=True)

        def s_wait(b):
            pltpu.make_async_copy(rowsl[b], acc.at[dst_v.at[0]],
                                  ssem[b]).wait()

        for t in range(_NB):
            pltpu.sync_copy(src_hbm.at[w].at[t], src_v)
            pltpu.sync_copy(dst_hbm.at[w].at[t], dst_v)
            g_issue(0, 0)
            g_issue(1, 1)
            g_wait(0); g_issue(2, 2); s_issue(0, 0)
            g_wait(1); g_issue(3, 3); s_issue(1, 1)
            g_wait(2); s_wait(0); g_issue(4, 0); s_issue(2, 2)
            g_wait(3); s_wait(1); g_issue(5, 1); s_issue(3, 3)

            def steady(q, carry):
                for u in range(4):
                    j = 4 * q + u
                    bn = (u + 2) % 4
                    g_wait(u)
                    s_wait(bn)
                    g_issue(j + 2, bn)
                    s_issue(j, u)
                return carry

            lax.fori_loop(1, _NG - 1, steady, 0)
            g_wait(0); s_wait(2); g_issue(_NCHB - 2, 2); s_issue(_NCHB - 4, 0)
            g_wait(1); s_wait(3); g_issue(_NCHB - 1, 3); s_issue(_NCHB - 3, 1)
            g_wait(2); s_issue(_NCHB - 2, 2)
            g_wait(3); s_issue(_NCHB - 1, 3)
            s_wait(0); s_wait(1); s_wait(2); s_wait(3)
        plsc.subcore_barrier()
        pltpu.sync_copy(acc.at[pl.ds(s * _RT, _RT)], out_hbm.at[c].at[s])

    return k(y, src_r, dst_r, zrows).reshape(_NC, _N, _D)


def _dis_block(degp_blk):
    deg = degp_blk[0][:, 0:1] + degp_blk[1][:, 0:1] + 1.0
    return 1.0 / jnp.sqrt(deg)


def _tc_y1(x, W1, degp):
    def body(x_ref, w_ref, degp_ref, y_ref):
        dis = _dis_block(degp_ref)
        xw = jnp.dot(x_ref[...], w_ref[...], preferred_element_type=jnp.float32)
        y_ref[...] = xw * dis

    return pl.pallas_call(
        body,
        grid=(_GR,),
        in_specs=[
            pl.BlockSpec((_BR, _D), lambda i: (i, 0)),
            pl.BlockSpec((_D, _D), lambda i: (0, 0)),
            pl.BlockSpec((_NC, _BR, _DW), lambda i: (0, i, 0)),
        ],
        out_specs=pl.BlockSpec((_BR, _D), lambda i: (i, 0)),
        out_shape=jax.ShapeDtypeStruct((_N, _D), jnp.float32),
    )(x, W1, degp)


def _tc_mid(S1, y1, degp, W2, b1r):
    def body(s_ref, y1_ref, degp_ref, w2_ref, b1_ref, y2_ref):
        dis = _dis_block(degp_ref)
        t = s_ref[0] + s_ref[1] + y1_ref[...]
        h1 = jnp.maximum(dis * t + b1_ref[...], 0.0)
        y2_ref[...] = jnp.dot(h1, w2_ref[...],
                              preferred_element_type=jnp.float32) * dis

    return pl.pallas_call(
        body,
        grid=(_GR,),
        in_specs=[
            pl.BlockSpec((_NC, _BR, _D), lambda i: (0, i, 0)),
            pl.BlockSpec((_BR, _D), lambda i: (i, 0)),
            pl.BlockSpec((_NC, _BR, _DW), lambda i: (0, i, 0)),
            pl.BlockSpec((_D, _D), lambda i: (0, 0)),
            pl.BlockSpec((1, _D), lambda i: (0, 0)),
        ],
        out_specs=pl.BlockSpec((_BR, _D), lambda i: (i, 0)),
        out_shape=jax.ShapeDtypeStruct((_N, _D), jnp.float32),
    )(S1, y1, degp, W2, b1r)


def _tc_final(S2, y2, degp, batch_r, b2r, FW1, Fb1r, FW2r, Fb2r):
    def body(s_ref, y2_ref, degp_ref, batch_ref, b2_ref, fw1_ref, fb1_ref,
             fw2_ref, fb2_ref, out_ref, sums, cnt):
        i = pl.program_id(0)

        @pl.when(i == 0)
        def _():
            sums[...] = jnp.zeros_like(sums)
            cnt[...] = jnp.zeros_like(cnt)

        dis = _dis_block(degp_ref)
        t = s_ref[0] + s_ref[1] + y2_ref[...]
        h2 = jnp.maximum(dis * t + b2_ref[...], 0.0)
        b = batch_ref[0, 0, :]
        gid = lax.broadcasted_iota(jnp.int32, (_G, _BR), 0)
        onehot = (gid == b[None, :]).astype(jnp.float32)
        sums[...] += jnp.dot(onehot, h2, preferred_element_type=jnp.float32, precision=lax.Precision.HIGHEST)
        cnt[...] += jnp.sum(onehot, axis=1, keepdims=True)

        @pl.when(i == _GR - 1)
        def _():
            pooled = sums[...] / jnp.maximum(cnt[...], 1.0)
            hid = jnp.maximum(
                jnp.dot(pooled, fw1_ref[...],
                        preferred_element_type=jnp.float32) + fb1_ref[...],
                0.0)
            hidb = hid.astype(jnp.bfloat16).astype(jnp.float32)
            fw2b = fw2_ref[...].astype(jnp.bfloat16).astype(jnp.float32)
            out_ref[...] = (jnp.sum(hidb * fw2b, axis=1, keepdims=True)
                            + fb2_ref[...])

    return pl.pallas_call(
        body,
        grid=(_GR,),
        in_specs=[
            pl.BlockSpec((_NC, _BR, _D), lambda i: (0, i, 0)),
            pl.BlockSpec((_BR, _D), lambda i: (i, 0)),
            pl.BlockSpec((_NC, _BR, _DW), lambda i: (0, i, 0)),
            pl.BlockSpec((1, 1, _BR), lambda i: (i, 0, 0)),
            pl.BlockSpec((1, _D), lambda i: (0, 0)),
            pl.BlockSpec((_D, _G), lambda i: (0, 0)),
            pl.BlockSpec((1, _G), lambda i: (0, 0)),
            pl.BlockSpec((1, _G), lambda i: (0, 0)),
            pl.BlockSpec((1, 1), lambda i: (0, 0)),
        ],
        out_specs=pl.BlockSpec((_G, 1), lambda i: (0, 0)),
        out_shape=jax.ShapeDtypeStruct((_G, 1), jnp.float32),
        scratch_shapes=[
            pltpu.VMEM((_G, _D), jnp.float32),
            pltpu.VMEM((_G, _D), jnp.float32),
        ],
    )(S2, y2, degp, batch_r, b2r, FW1, Fb1r, FW2r, Fb2r)


def kernel(x, edge_index, batch, W1, b1, W2, b2, FW1, Fb1, FW2, Fb2):
    src_r = edge_index[0].reshape(_NW, _NB, _NCHB, _K)
    dst_r = edge_index[1].reshape(_NW, _NB, _NCHB, _K)
    dst_deg = edge_index[1].reshape(_NW, _NCH, _K)
    batch_r = batch.reshape(_GR, 1, _BR)
    zrows = jnp.zeros((_RT, _D), jnp.float32)
    ones_rows = jnp.ones((_K, _D), jnp.float32)
    b1r = b1.reshape(1, _D)
    b2r = b2.reshape(1, _D)
    Fb1r = Fb1.reshape(1, _G)
    FW2r = FW2.reshape(1, _G)
    Fb2r = Fb2.reshape(1, 1)

    degp = _sc_degree(dst_deg, ones_rows, zrows)[:, :, :_DW]
    y1 = _tc_y1(x, W1, degp)
    S1 = _sc_msgpass(y1, src_r, dst_r, zrows)
    y2 = _tc_mid(S1, y1, degp, W2, b1r)
    S2 = _sc_msgpass(y2, src_r, dst_r, zrows)
    out = _tc_final(S2, y2, degp, batch_r, b2r, FW1, Fb1r, FW2r, Fb2r)
    return jnp.squeeze(out, axis=-1)
